# baseline (speedup 1.0000x reference)
import jax
import jax.numpy as jnp
from jax.experimental import pallas as pl
from jax.experimental.pallas import tpu as pltpu

_HIDDEN = 2048
_INTER = 2048
_TOKENS = 32
_BJ = 256
_NJ = _INTER // _BJ
_DEPTH = 3


def _copies(j, slot, wg_hbm, wu_hbm, wd_hbm, wg_buf, wu_buf, wd_buf, sems):
    return (
        pltpu.make_async_copy(wg_hbm.at[:, pl.ds(j * _BJ, _BJ)],
                              wg_buf.at[slot], sems.at[slot, 0]),
        pltpu.make_async_copy(wu_hbm.at[:, pl.ds(j * _BJ, _BJ)],
                              wu_buf.at[slot], sems.at[slot, 1]),
        pltpu.make_async_copy(wd_hbm.at[pl.ds(j * _BJ, _BJ), :],
                              wd_buf.at[slot], sems.at[slot, 2]),
    )


def _mlp_kernel(x_ref, wg_hbm, wu_hbm, wd_hbm, o_ref,
                wg_buf, wu_buf, wd_buf, sems):
    for j in range(min(_DEPTH, _NJ)):
        for c in _copies(j, j % _DEPTH, wg_hbm, wu_hbm, wd_hbm,
                         wg_buf, wu_buf, wd_buf, sems):
            c.start()

    x = x_ref[...]
    for j in range(_NJ):
        slot = j % _DEPTH
        for c in _copies(j, slot, wg_hbm, wu_hbm, wd_hbm,
                         wg_buf, wu_buf, wd_buf, sems):
            c.wait()
        gate = jnp.dot(x, wg_buf[slot], preferred_element_type=jnp.float32)
        up = jnp.dot(x, wu_buf[slot], preferred_element_type=jnp.float32)
        act = gate * jax.nn.sigmoid(gate) * up
        contrib = jnp.dot(act, wd_buf[slot],
                          preferred_element_type=jnp.float32)
        if j + _DEPTH < _NJ:
            for c in _copies(j + _DEPTH, slot, wg_hbm, wu_hbm, wd_hbm,
                             wg_buf, wu_buf, wd_buf, sems):
                c.start()
        if j == 0:
            o_ref[...] = contrib
        else:
            o_ref[...] += contrib


def kernel(x, W_gate, W_up, W_down):
    return pl.pallas_call(
        _mlp_kernel,
        in_specs=[
            pl.BlockSpec(memory_space=pltpu.MemorySpace.VMEM),
            pl.BlockSpec(memory_space=pltpu.MemorySpace.HBM),
            pl.BlockSpec(memory_space=pltpu.MemorySpace.HBM),
            pl.BlockSpec(memory_space=pltpu.MemorySpace.HBM),
        ],
        out_specs=pl.BlockSpec(memory_space=pltpu.MemorySpace.VMEM),
        out_shape=jax.ShapeDtypeStruct((_TOKENS, _HIDDEN), jnp.float32),
        scratch_shapes=[
            pltpu.VMEM((_DEPTH, _HIDDEN, _BJ), jnp.float32),
            pltpu.VMEM((_DEPTH, _HIDDEN, _BJ), jnp.float32),
            pltpu.VMEM((_DEPTH, _BJ, _HIDDEN), jnp.float32),
            pltpu.SemaphoreType.DMA((_DEPTH, 3)),
        ],
    )(x, W_gate, W_up, W_down)


# speedup vs baseline: 1.0676x; 1.0676x over previous
import jax
import jax.numpy as jnp
from jax.experimental import pallas as pl

_HIDDEN = 2048
_INTER = 2048
_TOKENS = 32
_BJ = 256
_Q = 4
_HQ = _HIDDEN // _Q
_DQ = _BJ // _Q


def _mlp_kernel(x_ref, *refs):
    wg = refs[0:_Q]
    wu = refs[_Q:2 * _Q]
    wd = refs[2 * _Q:3 * _Q]
    o_ref = refs[3 * _Q]
    j = pl.program_id(0)

    x = x_ref[...]
    gate = jnp.dot(x[:, 0:_HQ], wg[0][...], preferred_element_type=jnp.float32)
    up = jnp.dot(x[:, 0:_HQ], wu[0][...], preferred_element_type=jnp.float32)
    for q in range(1, _Q):
        xq = x[:, q * _HQ:(q + 1) * _HQ]
        gate = gate + jnp.dot(xq, wg[q][...], preferred_element_type=jnp.float32)
        up = up + jnp.dot(xq, wu[q][...], preferred_element_type=jnp.float32)
    act = gate * jax.nn.sigmoid(gate) * up

    contrib = jnp.dot(act[:, 0:_DQ], wd[0][...],
                      preferred_element_type=jnp.float32)
    for q in range(1, _Q):
        contrib = contrib + jnp.dot(act[:, q * _DQ:(q + 1) * _DQ], wd[q][...],
                                    preferred_element_type=jnp.float32)

    @pl.when(j == 0)
    def _init():
        o_ref[...] = contrib

    @pl.when(j > 0)
    def _acc():
        o_ref[...] += contrib


def kernel(x, W_gate, W_up, W_down):
    wspecs = [pl.BlockSpec((_HQ, _BJ), lambda j, q=q: (q, j))
              for q in range(_Q)]
    dspecs = [pl.BlockSpec((_DQ, _HIDDEN), lambda j, q=q: (_Q * j + q, 0))
              for q in range(_Q)]
    return pl.pallas_call(
        _mlp_kernel,
        grid=(_INTER // _BJ,),
        in_specs=([pl.BlockSpec((_TOKENS, _HIDDEN), lambda j: (0, 0))]
                  + wspecs + wspecs + dspecs),
        out_specs=pl.BlockSpec((_TOKENS, _HIDDEN), lambda j: (0, 0)),
        out_shape=jax.ShapeDtypeStruct((_TOKENS, _HIDDEN), jnp.float32),
    )(x, *([W_gate] * _Q), *([W_up] * _Q), *([W_down] * _Q))
